# baseline (speedup 1.0000x reference)
import functools

import jax
import jax.numpy as jnp
from jax import lax
from jax.experimental import pallas as pl
from jax.experimental.pallas import tpu as pltpu
from jax.experimental.pallas import tpu_sc as plsc

N = 10000
E = 320000
F_IN = 128
H1 = 64
NL = 16

NTILES = 32
EPT = E // NTILES
BLK = 128
NBLK = 80
EPT_PAD = NBLK * BLK
R = 10240
W1PAD = 80
W2PAD = 32


def _phase1_body(x_ref, w1_ref, a1s_ref, a1d_ref, g1_ref, bt1_ref,
                 table_ref, s_ref, d_ref, m_ref):
    xv = x_ref[...]
    mu = jnp.mean(xv, axis=0, keepdims=True)
    var = jnp.mean((xv - mu) ** 2, axis=0, keepdims=True)
    xn = (xv - mu) * lax.rsqrt(var + 1e-5) * g1_ref[...] + bt1_ref[...]
    hw = lax.dot_general(xn, w1_ref[...], (((1,), (0,)), ((), ())),
                         preferred_element_type=jnp.float32)
    s = lax.dot_general(hw, a1s_ref[...], (((1,), (0,)), ((), ())),
                        preferred_element_type=jnp.float32)
    d = lax.dot_general(hw, a1d_ref[...], (((1,), (0,)), ((), ())),
                        preferred_element_type=jnp.float32)
    table_ref[...] = hw
    s_ref[...] = s
    d_ref[...] = d
    m = jnp.max(s) + jnp.max(d)
    m = jnp.maximum(m, 0.2 * m)
    m_ref[...] = jnp.full((1, 16), m, jnp.float32)


def _phase2_body(slab_ref, b1_ref, g2_ref, bt2_ref, w2_ref, a2s_ref, a2d_ref,
                 table_ref, s_ref, d_ref, m_ref):
    acc = slab_ref[0, :N, :] + slab_ref[1, :N, :]
    num = acc[:, :H1]
    den = acc[:, H1:H1 + 1]
    h = jnp.tanh(num / (den + 1e-16) + b1_ref[...])
    mu = jnp.mean(h, axis=0, keepdims=True)
    var = jnp.mean((h - mu) ** 2, axis=0, keepdims=True)
    hn = (h - mu) * lax.rsqrt(var + 1e-5) * g2_ref[...] + bt2_ref[...]
    hw = lax.dot_general(hn, w2_ref[...], (((1,), (0,)), ((), ())),
                         preferred_element_type=jnp.float32)
    s = lax.dot_general(hw, a2s_ref[...], (((1,), (0,)), ((), ())),
                        preferred_element_type=jnp.float32)
    d = lax.dot_general(hw, a2d_ref[...], (((1,), (0,)), ((), ())),
                        preferred_element_type=jnp.float32)
    table_ref[...] = hw
    s_ref[...] = s
    d_ref[...] = d
    m = jnp.max(s) + jnp.max(d)
    m = jnp.maximum(m, 0.2 * m)
    m_ref[...] = jnp.full((1, 16), m, jnp.float32)


def _phase3_body(slab_ref, b2_ref, out_ref):
    acc = slab_ref[0, :N, :] + slab_ref[1, :N, :]
    num = acc[:, :NL]
    den = acc[:, NL:NL + 1]
    h = jnp.tanh(num / (den + 1e-16) + b2_ref[...])
    mx = jnp.max(h, axis=1, keepdims=True)
    lse = jnp.log(jnp.sum(jnp.exp(h - mx), axis=1, keepdims=True))
    out_ref[...] = h - mx - lse


_phase1 = pl.pallas_call(
    _phase1_body,
    out_shape=[
        jax.ShapeDtypeStruct((N, H1), jnp.float32),
        jax.ShapeDtypeStruct((N, 1), jnp.float32),
        jax.ShapeDtypeStruct((N, 1), jnp.float32),
        jax.ShapeDtypeStruct((1, 16), jnp.float32),
    ],
)

_phase2 = pl.pallas_call(
    _phase2_body,
    out_shape=[
        jax.ShapeDtypeStruct((N, NL), jnp.float32),
        jax.ShapeDtypeStruct((N, 1), jnp.float32),
        jax.ShapeDtypeStruct((N, 1), jnp.float32),
        jax.ShapeDtypeStruct((1, 16), jnp.float32),
    ],
)

_phase3 = pl.pallas_call(
    _phase3_body,
    out_shape=jax.ShapeDtypeStruct((N, NL), jnp.float32),
)


def _make_edge_kernel(width: int, gwidth: int):
    ngv = gwidth // 16
    nvec = width // 16
    rows_per_tile = R // 16
    mesh = plsc.VectorSubcoreMesh(core_axis_name="c", subcore_axis_name="s")

    @functools.partial(
        pl.kernel,
        out_type=jax.ShapeDtypeStruct((2, R, width), jnp.float32),
        mesh=mesh,
        compiler_params=pltpu.CompilerParams(needs_layout_passes=False,
                                             use_tc_tiling_on_sc=False),
        scratch_types=[
            pltpu.VMEM((N,), jnp.float32),
            pltpu.VMEM((N,), jnp.float32),
            pltpu.VMEM((16,), jnp.float32),
            pltpu.VMEM((NBLK, BLK), jnp.int32),
            pltpu.VMEM((NBLK, BLK), jnp.int32),
            pltpu.VMEM((BLK,), jnp.float32),
            [pltpu.VMEM((BLK, gwidth), jnp.float32) for _ in range(2)],
            [pltpu.VMEM((BLK, width), jnp.float32) for _ in range(2)],
            pltpu.VMEM_SHARED((R, width), jnp.float32),
            [pltpu.SemaphoreType.DMA for _ in range(2)],
            [pltpu.SemaphoreType.DMA for _ in range(2)],
        ],
    )
    def ek(table_hbm, s_hbm, d_hbm, m_hbm, src_hbm, dst_hbm, out_hbm,
           s_v, d_v, m_v, srcb, dstb, wbuf, rows, obuf, accum, gsem, ssem):
        c = lax.axis_index("c")
        sub = lax.axis_index("s")
        wid = sub * 2 + c

        pltpu.sync_copy(s_hbm, s_v)
        pltpu.sync_copy(d_hbm, d_v)
        pltpu.sync_copy(m_hbm, m_v)
        pltpu.sync_copy(src_hbm.at[pl.ds(wid * NBLK, NBLK)], srcb)
        pltpu.sync_copy(dst_hbm.at[pl.ds(wid * NBLK, NBLK)], dstb)

        zeros16 = jnp.zeros((16,), jnp.float32)

        def zero_body(i, _):
            for j in range(nvec):
                obuf[0][i, pl.ds(j * 16, 16)] = zeros16
                obuf[1][i, pl.ds(j * 16, 16)] = zeros16
            return _

        lax.fori_loop(0, BLK, zero_body, None)

        def zcopy_body(k, _):
            pltpu.sync_copy(obuf[0], accum.at[pl.ds(sub * rows_per_tile + k * BLK, BLK)])
            return _

        lax.fori_loop(0, rows_per_tile // BLK, zcopy_body, None)
        plsc.subcore_barrier()

        mv = m_v[...]
        lane = lax.iota(jnp.int32, 16)

        def gather_of(b, u):
            return (
                pltpu.make_async_copy(table_hbm.at[srcb.at[b, pl.ds(0, 64)]],
                                      rows[u].at[pl.ds(0, 64)], gsem[u]),
                pltpu.make_async_copy(table_hbm.at[srcb.at[b, pl.ds(64, 64)]],
                                      rows[u].at[pl.ds(64, 64)], gsem[u]),
            )

        def scatter_of(b, v):
            return pltpu.make_async_copy(obuf[v], accum.at[dstb.at[b]], ssem[v])

        for g in gather_of(0, 0) + gather_of(1, 1):
            g.start()

        def half(b, u):
            def w_body(g, _):
                si = srcb[b, pl.ds(g * 16, 16)]
                di = dstb[b, pl.ds(g * 16, 16)]
                t = plsc.load_gather(s_v, [si]) + plsc.load_gather(d_v, [di])
                t = jnp.maximum(t, 0.2 * t)
                wbuf[pl.ds(g * 16, 16)] = jnp.exp(t - mv)
                return _

            lax.fori_loop(0, BLK // 16, w_body, None)
            for g in gather_of(b, u):
                g.wait()

            @pl.when(b >= 2)
            def _drain():
                scatter_of(b - 2, u).wait()

            def wcol_body(g, _):
                w16 = wbuf[pl.ds(g * 16, 16)]
                plsc.store_scatter(obuf[u], [g * 16 + lane,
                                             jnp.zeros((16,), jnp.int32) + gwidth], w16)
                return _

            lax.fori_loop(0, BLK // 16, wcol_body, None)

            def mul_body(i, _):
                for uu in range(4):
                    e = i * 4 + uu
                    wv = plsc.load_gather(wbuf, [jnp.zeros((16,), jnp.int32) + e])
                    for j in range(ngv):
                        obuf[u][e, pl.ds(j * 16, 16)] = rows[u][e, pl.ds(j * 16, 16)] * wv
                return _

            lax.fori_loop(0, BLK // 4, mul_body, None)

            @pl.when(b + 2 < NBLK)
            def _prefetch():
                for g in gather_of(b + 2, u):
                    g.start()

            pltpu.async_copy(obuf[u], accum.at[dstb.at[b]], ssem[u], add=True)

        def loop_body(i, _):
            half(2 * i, 0)
            half(2 * i + 1, 1)
            return _

        lax.fori_loop(0, NBLK // 2, loop_body, None)
        scatter_of(NBLK - 2, 0).wait()
        scatter_of(NBLK - 1, 1).wait()
        plsc.subcore_barrier()

        for k in range(rows_per_tile // BLK):
            r0 = sub * rows_per_tile + k * BLK
            pltpu.sync_copy(accum.at[pl.ds(r0, BLK)], out_hbm.at[c, pl.ds(r0, BLK)])

    return ek


_edge1 = _make_edge_kernel(W1PAD, H1)
_edge2 = _make_edge_kernel(W2PAD, NL)


def kernel(x, edge_index_all, W1, a1s, a1d, b1, g1, bt1, W2, a2s, a2d, b2, g2, bt2):
    src = jnp.pad(edge_index_all[0].reshape(NTILES, EPT),
                  ((0, 0), (0, EPT_PAD - EPT))).reshape(NTILES * NBLK, BLK)
    dst = jnp.pad(edge_index_all[1].reshape(NTILES, EPT),
                  ((0, 0), (0, EPT_PAD - EPT)),
                  constant_values=N).reshape(NTILES * NBLK, BLK)

    table1, s1, d1, m1 = _phase1(
        x, W1, a1s.reshape(H1, 1), a1d.reshape(H1, 1),
        g1.reshape(1, F_IN), bt1.reshape(1, F_IN))

    slab1 = _edge1(table1, s1.reshape(N), d1.reshape(N), m1.reshape(16), src, dst)

    table2, s2, d2, m2 = _phase2(
        slab1, b1.reshape(1, H1), g2.reshape(1, H1), bt2.reshape(1, H1),
        W2, a2s.reshape(NL, 1), a2d.reshape(NL, 1))

    slab2 = _edge2(table2, s2.reshape(N), d2.reshape(N), m2.reshape(16), src, dst)

    return _phase3(slab2, b2.reshape(1, NL))


# speedup vs baseline: 1.0519x; 1.0519x over previous
import functools

import jax
import jax.numpy as jnp
from jax import lax
from jax.experimental import pallas as pl
from jax.experimental.pallas import tpu as pltpu
from jax.experimental.pallas import tpu_sc as plsc

N = 10000
E = 320000
F_IN = 128
H1 = 64
NL = 16

NTILES = 32
EPT = E // NTILES
BLK = 128
NBLK = 80
EPT_PAD = NBLK * BLK
R = 10240
W1PAD = 80
W2PAD = 32


def _phase1_body(x_ref, w1_ref, a1s_ref, a1d_ref, g1_ref, bt1_ref,
                 table_ref, s_ref, d_ref, m_ref):
    xv = x_ref[...]
    mu = jnp.mean(xv, axis=0, keepdims=True)
    var = jnp.mean((xv - mu) ** 2, axis=0, keepdims=True)
    xn = (xv - mu) * lax.rsqrt(var + 1e-5) * g1_ref[...] + bt1_ref[...]
    hw = lax.dot_general(xn, w1_ref[...], (((1,), (0,)), ((), ())),
                         preferred_element_type=jnp.float32)
    s = lax.dot_general(hw, a1s_ref[...], (((1,), (0,)), ((), ())),
                        preferred_element_type=jnp.float32)
    d = lax.dot_general(hw, a1d_ref[...], (((1,), (0,)), ((), ())),
                        preferred_element_type=jnp.float32)
    table_ref[...] = hw
    s_ref[...] = s
    d_ref[...] = d
    m = jnp.max(s) + jnp.max(d)
    m = jnp.maximum(m, 0.2 * m)
    m_ref[...] = jnp.full((1, 16), m, jnp.float32)


def _phase2_body(slab_ref, b1_ref, g2_ref, bt2_ref, w2_ref, a2s_ref, a2d_ref,
                 table_ref, s_ref, d_ref, m_ref):
    acc = slab_ref[0, :N, :] + slab_ref[1, :N, :]
    num = acc[:, :H1]
    den = acc[:, H1:H1 + 1]
    h = jnp.tanh(num / (den + 1e-16) + b1_ref[...])
    mu = jnp.mean(h, axis=0, keepdims=True)
    var = jnp.mean((h - mu) ** 2, axis=0, keepdims=True)
    hn = (h - mu) * lax.rsqrt(var + 1e-5) * g2_ref[...] + bt2_ref[...]
    hw = lax.dot_general(hn, w2_ref[...], (((1,), (0,)), ((), ())),
                         preferred_element_type=jnp.float32)
    s = lax.dot_general(hw, a2s_ref[...], (((1,), (0,)), ((), ())),
                        preferred_element_type=jnp.float32)
    d = lax.dot_general(hw, a2d_ref[...], (((1,), (0,)), ((), ())),
                        preferred_element_type=jnp.float32)
    table_ref[...] = hw
    s_ref[...] = s
    d_ref[...] = d
    m = jnp.max(s) + jnp.max(d)
    m = jnp.maximum(m, 0.2 * m)
    m_ref[...] = jnp.full((1, 16), m, jnp.float32)


def _phase3_body(slab_ref, b2_ref, out_ref):
    acc = slab_ref[0, :N, :] + slab_ref[1, :N, :]
    num = acc[:, :NL]
    den = acc[:, NL:NL + 1]
    h = jnp.tanh(num / (den + 1e-16) + b2_ref[...])
    mx = jnp.max(h, axis=1, keepdims=True)
    lse = jnp.log(jnp.sum(jnp.exp(h - mx), axis=1, keepdims=True))
    out_ref[...] = h - mx - lse


_phase1 = pl.pallas_call(
    _phase1_body,
    out_shape=[
        jax.ShapeDtypeStruct((N, H1), jnp.float32),
        jax.ShapeDtypeStruct((N, 1), jnp.float32),
        jax.ShapeDtypeStruct((N, 1), jnp.float32),
        jax.ShapeDtypeStruct((1, 16), jnp.float32),
    ],
)

_phase2 = pl.pallas_call(
    _phase2_body,
    out_shape=[
        jax.ShapeDtypeStruct((N, NL), jnp.float32),
        jax.ShapeDtypeStruct((N, 1), jnp.float32),
        jax.ShapeDtypeStruct((N, 1), jnp.float32),
        jax.ShapeDtypeStruct((1, 16), jnp.float32),
    ],
)

_phase3 = pl.pallas_call(
    _phase3_body,
    out_shape=jax.ShapeDtypeStruct((N, NL), jnp.float32),
)


def _make_edge_kernel(width: int, gwidth: int, table_in_spmem: bool = False):
    ngv = gwidth // 16
    nvec = width // 16
    rows_per_tile = R // 16
    mesh = plsc.VectorSubcoreMesh(core_axis_name="c", subcore_axis_name="s")

    @functools.partial(
        pl.kernel,
        out_type=jax.ShapeDtypeStruct((2, R, width), jnp.float32),
        mesh=mesh,
        compiler_params=pltpu.CompilerParams(needs_layout_passes=False,
                                             use_tc_tiling_on_sc=False),
        scratch_types=[
            pltpu.VMEM((N,), jnp.float32),
            pltpu.VMEM((N,), jnp.float32),
            pltpu.VMEM((16,), jnp.float32),
            pltpu.VMEM((NBLK, BLK), jnp.int32),
            pltpu.VMEM((NBLK, BLK), jnp.int32),
            pltpu.VMEM((BLK,), jnp.float32),
            [pltpu.VMEM((BLK, gwidth), jnp.float32) for _ in range(2)],
            [pltpu.VMEM((BLK, width), jnp.float32) for _ in range(2)],
            pltpu.VMEM_SHARED((R, width), jnp.float32),
            (pltpu.VMEM_SHARED((N, gwidth), jnp.float32)
             if table_in_spmem else pltpu.VMEM((16,), jnp.float32)),
            [pltpu.SemaphoreType.DMA for _ in range(2)],
            [pltpu.SemaphoreType.DMA for _ in range(2)],
        ],
    )
    def ek(table_hbm, s_hbm, d_hbm, m_hbm, src_hbm, dst_hbm, out_hbm,
           s_v, d_v, m_v, srcb, dstb, wbuf, rows, obuf, accum, table_s, gsem, ssem):
        c = lax.axis_index("c")
        sub = lax.axis_index("s")
        wid = sub * 2 + c

        pltpu.sync_copy(s_hbm, s_v)
        pltpu.sync_copy(d_hbm, d_v)
        pltpu.sync_copy(m_hbm, m_v)
        pltpu.sync_copy(src_hbm.at[pl.ds(wid * NBLK, NBLK)], srcb)
        pltpu.sync_copy(dst_hbm.at[pl.ds(wid * NBLK, NBLK)], dstb)

        zeros16 = jnp.zeros((16,), jnp.float32)

        def zero_body(i, _):
            for j in range(nvec):
                obuf[0][i, pl.ds(j * 16, 16)] = zeros16
                obuf[1][i, pl.ds(j * 16, 16)] = zeros16
            return _

        lax.fori_loop(0, BLK, zero_body, None)

        def zcopy_body(k, _):
            pltpu.sync_copy(obuf[0], accum.at[pl.ds(sub * rows_per_tile + k * BLK, BLK)])
            return _

        lax.fori_loop(0, rows_per_tile // BLK, zcopy_body, None)
        if table_in_spmem:
            npt = N // 16
            pltpu.sync_copy(table_hbm.at[pl.ds(sub * npt, npt)],
                            table_s.at[pl.ds(sub * npt, npt)])
        plsc.subcore_barrier()

        mv = m_v[...]
        lane = lax.iota(jnp.int32, 16)

        table_ref = table_s if table_in_spmem else table_hbm

        def gather_of(b, u):
            return (
                pltpu.make_async_copy(table_ref.at[srcb.at[b, pl.ds(0, 64)]],
                                      rows[u].at[pl.ds(0, 64)], gsem[u]),
                pltpu.make_async_copy(table_ref.at[srcb.at[b, pl.ds(64, 64)]],
                                      rows[u].at[pl.ds(64, 64)], gsem[u]),
            )

        def scatter_of(b, v):
            return pltpu.make_async_copy(obuf[v], accum.at[dstb.at[b]], ssem[v])

        for g in gather_of(0, 0) + gather_of(1, 1):
            g.start()

        def half(b, u):
            def w_body(g, _):
                si = srcb[b, pl.ds(g * 16, 16)]
                di = dstb[b, pl.ds(g * 16, 16)]
                t = plsc.load_gather(s_v, [si]) + plsc.load_gather(d_v, [di])
                t = jnp.maximum(t, 0.2 * t)
                wbuf[pl.ds(g * 16, 16)] = jnp.exp(t - mv)
                return _

            lax.fori_loop(0, BLK // 16, w_body, None)
            for g in gather_of(b, u):
                g.wait()

            @pl.when(b >= 2)
            def _drain():
                scatter_of(b - 2, u).wait()

            def wcol_body(g, _):
                w16 = wbuf[pl.ds(g * 16, 16)]
                plsc.store_scatter(obuf[u], [g * 16 + lane,
                                             jnp.zeros((16,), jnp.int32) + gwidth], w16)
                return _

            lax.fori_loop(0, BLK // 16, wcol_body, None)

            def mul_body(i, _):
                for uu in range(4):
                    e = i * 4 + uu
                    wv = plsc.load_gather(wbuf, [jnp.zeros((16,), jnp.int32) + e])
                    for j in range(ngv):
                        obuf[u][e, pl.ds(j * 16, 16)] = rows[u][e, pl.ds(j * 16, 16)] * wv
                return _

            lax.fori_loop(0, BLK // 4, mul_body, None)

            @pl.when(b + 2 < NBLK)
            def _prefetch():
                for g in gather_of(b + 2, u):
                    g.start()

            pltpu.async_copy(obuf[u], accum.at[dstb.at[b]], ssem[u], add=True)

        def loop_body(i, _):
            half(2 * i, 0)
            half(2 * i + 1, 1)
            return _

        lax.fori_loop(0, NBLK // 2, loop_body, None)
        scatter_of(NBLK - 2, 0).wait()
        scatter_of(NBLK - 1, 1).wait()
        plsc.subcore_barrier()

        for k in range(rows_per_tile // BLK):
            r0 = sub * rows_per_tile + k * BLK
            pltpu.sync_copy(accum.at[pl.ds(r0, BLK)], out_hbm.at[c, pl.ds(r0, BLK)])

    return ek


_edge1 = _make_edge_kernel(W1PAD, H1)
_edge2 = _make_edge_kernel(W2PAD, NL, table_in_spmem=True)


def kernel(x, edge_index_all, W1, a1s, a1d, b1, g1, bt1, W2, a2s, a2d, b2, g2, bt2):
    src = jnp.pad(edge_index_all[0].reshape(NTILES, EPT),
                  ((0, 0), (0, EPT_PAD - EPT))).reshape(NTILES * NBLK, BLK)
    dst = jnp.pad(edge_index_all[1].reshape(NTILES, EPT),
                  ((0, 0), (0, EPT_PAD - EPT)),
                  constant_values=N).reshape(NTILES * NBLK, BLK)

    table1, s1, d1, m1 = _phase1(
        x, W1, a1s.reshape(H1, 1), a1d.reshape(H1, 1),
        g1.reshape(1, F_IN), bt1.reshape(1, F_IN))

    slab1 = _edge1(table1, s1.reshape(N), d1.reshape(N), m1.reshape(16), src, dst)

    table2, s2, d2, m2 = _phase2(
        slab1, b1.reshape(1, H1), g2.reshape(1, H1), bt2.reshape(1, H1),
        W2, a2s.reshape(NL, 1), a2d.reshape(NL, 1))

    slab2 = _edge2(table2, s2.reshape(N), d2.reshape(N), m2.reshape(16), src, dst)

    return _phase3(slab2, b2.reshape(1, NL))


# speedup vs baseline: 1.0560x; 1.0039x over previous
import functools

import jax
import jax.numpy as jnp
from jax import lax
from jax.experimental import pallas as pl
from jax.experimental.pallas import tpu as pltpu
from jax.experimental.pallas import tpu_sc as plsc

N = 10000
E = 320000
F_IN = 128
H1 = 64
NL = 16

NTILES = 32
EPT = E // NTILES
BLK = 128
NBLK = 80
EPT_PAD = NBLK * BLK
R = 10240
W1PAD = 72
W2PAD = 24


def _phase1_body(x_ref, w1_ref, a1s_ref, a1d_ref, g1_ref, bt1_ref,
                 table_ref, s_ref, d_ref, m_ref):
    xv = x_ref[...]
    mu = jnp.mean(xv, axis=0, keepdims=True)
    var = jnp.mean((xv - mu) ** 2, axis=0, keepdims=True)
    xn = (xv - mu) * lax.rsqrt(var + 1e-5) * g1_ref[...] + bt1_ref[...]
    hw = lax.dot_general(xn, w1_ref[...], (((1,), (0,)), ((), ())),
                         preferred_element_type=jnp.float32)
    s = lax.dot_general(hw, a1s_ref[...], (((1,), (0,)), ((), ())),
                        preferred_element_type=jnp.float32)
    d = lax.dot_general(hw, a1d_ref[...], (((1,), (0,)), ((), ())),
                        preferred_element_type=jnp.float32)
    table_ref[...] = hw
    s_ref[...] = s
    d_ref[...] = d
    m = jnp.max(s) + jnp.max(d)
    m = jnp.maximum(m, 0.2 * m)
    m_ref[...] = jnp.full((1, 16), m, jnp.float32)


def _phase2_body(slab_ref, b1_ref, g2_ref, bt2_ref, w2_ref, a2s_ref, a2d_ref,
                 table_ref, s_ref, d_ref, m_ref):
    acc = slab_ref[0, :N, :] + slab_ref[1, :N, :]
    num = acc[:, :H1]
    den = acc[:, H1:H1 + 1]
    h = jnp.tanh(num / (den + 1e-16) + b1_ref[...])
    mu = jnp.mean(h, axis=0, keepdims=True)
    var = jnp.mean((h - mu) ** 2, axis=0, keepdims=True)
    hn = (h - mu) * lax.rsqrt(var + 1e-5) * g2_ref[...] + bt2_ref[...]
    hw = lax.dot_general(hn, w2_ref[...], (((1,), (0,)), ((), ())),
                         preferred_element_type=jnp.float32)
    s = lax.dot_general(hw, a2s_ref[...], (((1,), (0,)), ((), ())),
                        preferred_element_type=jnp.float32)
    d = lax.dot_general(hw, a2d_ref[...], (((1,), (0,)), ((), ())),
                        preferred_element_type=jnp.float32)
    table_ref[...] = hw
    s_ref[...] = s
    d_ref[...] = d
    m = jnp.max(s) + jnp.max(d)
    m = jnp.maximum(m, 0.2 * m)
    m_ref[...] = jnp.full((1, 16), m, jnp.float32)


def _phase3_body(slab_ref, b2_ref, out_ref):
    acc = slab_ref[0, :N, :] + slab_ref[1, :N, :]
    num = acc[:, :NL]
    den = acc[:, NL:NL + 1]
    h = jnp.tanh(num / (den + 1e-16) + b2_ref[...])
    mx = jnp.max(h, axis=1, keepdims=True)
    lse = jnp.log(jnp.sum(jnp.exp(h - mx), axis=1, keepdims=True))
    out_ref[...] = h - mx - lse


_phase1 = pl.pallas_call(
    _phase1_body,
    out_shape=[
        jax.ShapeDtypeStruct((N, H1), jnp.float32),
        jax.ShapeDtypeStruct((N, 1), jnp.float32),
        jax.ShapeDtypeStruct((N, 1), jnp.float32),
        jax.ShapeDtypeStruct((1, 16), jnp.float32),
    ],
)

_phase2 = pl.pallas_call(
    _phase2_body,
    out_shape=[
        jax.ShapeDtypeStruct((N, NL), jnp.float32),
        jax.ShapeDtypeStruct((N, 1), jnp.float32),
        jax.ShapeDtypeStruct((N, 1), jnp.float32),
        jax.ShapeDtypeStruct((1, 16), jnp.float32),
    ],
)

_phase3 = pl.pallas_call(
    _phase3_body,
    out_shape=jax.ShapeDtypeStruct((N, NL), jnp.float32),
)


def _make_edge_kernel(width: int, gwidth: int, table_in_spmem: bool = False):
    ngv = gwidth // 16
    nvec = width // 16
    rows_per_tile = R // 16
    mesh = plsc.VectorSubcoreMesh(core_axis_name="c", subcore_axis_name="s")

    @functools.partial(
        pl.kernel,
        out_type=jax.ShapeDtypeStruct((2, R, width), jnp.float32),
        mesh=mesh,
        compiler_params=pltpu.CompilerParams(needs_layout_passes=False,
                                             use_tc_tiling_on_sc=False),
        scratch_types=[
            pltpu.VMEM((N,), jnp.float32),
            pltpu.VMEM((N,), jnp.float32),
            pltpu.VMEM((16,), jnp.float32),
            pltpu.VMEM((NBLK, BLK), jnp.int32),
            pltpu.VMEM((NBLK, BLK), jnp.int32),
            pltpu.VMEM((BLK,), jnp.float32),
            [pltpu.VMEM((BLK, gwidth), jnp.float32) for _ in range(2)],
            [pltpu.VMEM((BLK, width), jnp.float32) for _ in range(2)],
            pltpu.VMEM_SHARED((R, width), jnp.float32),
            (pltpu.VMEM_SHARED((N, gwidth), jnp.float32)
             if table_in_spmem else pltpu.VMEM((16,), jnp.float32)),
            [pltpu.SemaphoreType.DMA for _ in range(2)],
            [pltpu.SemaphoreType.DMA for _ in range(2)],
        ],
    )
    def ek(table_hbm, s_hbm, d_hbm, m_hbm, src_hbm, dst_hbm, out_hbm,
           s_v, d_v, m_v, srcb, dstb, wbuf, rows, obuf, accum, table_s, gsem, ssem):
        c = lax.axis_index("c")
        sub = lax.axis_index("s")
        wid = sub * 2 + c

        pltpu.sync_copy(s_hbm, s_v)
        pltpu.sync_copy(d_hbm, d_v)
        pltpu.sync_copy(m_hbm, m_v)
        pltpu.sync_copy(src_hbm.at[pl.ds(wid * NBLK, NBLK)], srcb)
        pltpu.sync_copy(dst_hbm.at[pl.ds(wid * NBLK, NBLK)], dstb)

        zeros16 = jnp.zeros((16,), jnp.float32)

        zoffs = [j * 16 for j in range(width // 16)]
        if width % 16:
            zoffs.append(width - 16)

        def zero_body(i, _):
            for z in zoffs:
                obuf[0][i, pl.ds(z, 16)] = zeros16
                obuf[1][i, pl.ds(z, 16)] = zeros16
            return _

        lax.fori_loop(0, BLK, zero_body, None)

        def zcopy_body(k, _):
            pltpu.sync_copy(obuf[0], accum.at[pl.ds(sub * rows_per_tile + k * BLK, BLK)])
            return _

        lax.fori_loop(0, rows_per_tile // BLK, zcopy_body, None)
        if table_in_spmem:
            npt = N // 16
            pltpu.sync_copy(table_hbm.at[pl.ds(sub * npt, npt)],
                            table_s.at[pl.ds(sub * npt, npt)])
        plsc.subcore_barrier()

        mv = m_v[...]
        lane = lax.iota(jnp.int32, 16)

        table_ref = table_s if table_in_spmem else table_hbm

        def gather_of(b, u):
            return (
                pltpu.make_async_copy(table_ref.at[srcb.at[b, pl.ds(0, 64)]],
                                      rows[u].at[pl.ds(0, 64)], gsem[u]),
                pltpu.make_async_copy(table_ref.at[srcb.at[b, pl.ds(64, 64)]],
                                      rows[u].at[pl.ds(64, 64)], gsem[u]),
            )

        def scatter_of(b, v):
            return pltpu.make_async_copy(obuf[v], accum.at[dstb.at[b]], ssem[v])

        for g in gather_of(0, 0) + gather_of(1, 1):
            g.start()

        def half(b, u):
            def w_body(g, _):
                si = srcb[b, pl.ds(g * 16, 16)]
                di = dstb[b, pl.ds(g * 16, 16)]
                t = plsc.load_gather(s_v, [si]) + plsc.load_gather(d_v, [di])
                t = jnp.maximum(t, 0.2 * t)
                wbuf[pl.ds(g * 16, 16)] = jnp.exp(t - mv)
                return _

            lax.fori_loop(0, BLK // 16, w_body, None)
            for g in gather_of(b, u):
                g.wait()

            @pl.when(b >= 2)
            def _drain():
                scatter_of(b - 2, u).wait()

            def wcol_body(g, _):
                w16 = wbuf[pl.ds(g * 16, 16)]
                plsc.store_scatter(obuf[u], [g * 16 + lane,
                                             jnp.zeros((16,), jnp.int32) + gwidth], w16)
                return _

            lax.fori_loop(0, BLK // 16, wcol_body, None)

            def mul_body(i, _):
                for uu in range(4):
                    e = i * 4 + uu
                    wv = plsc.load_gather(wbuf, [jnp.zeros((16,), jnp.int32) + e])
                    for j in range(ngv):
                        obuf[u][e, pl.ds(j * 16, 16)] = rows[u][e, pl.ds(j * 16, 16)] * wv
                return _

            lax.fori_loop(0, BLK // 4, mul_body, None)

            @pl.when(b + 2 < NBLK)
            def _prefetch():
                for g in gather_of(b + 2, u):
                    g.start()

            pltpu.async_copy(obuf[u], accum.at[dstb.at[b]], ssem[u], add=True)

        def loop_body(i, _):
            half(2 * i, 0)
            half(2 * i + 1, 1)
            return _

        lax.fori_loop(0, NBLK // 2, loop_body, None)
        scatter_of(NBLK - 2, 0).wait()
        scatter_of(NBLK - 1, 1).wait()
        plsc.subcore_barrier()

        for k in range(rows_per_tile // BLK):
            r0 = sub * rows_per_tile + k * BLK
            pltpu.sync_copy(accum.at[pl.ds(r0, BLK)], out_hbm.at[c, pl.ds(r0, BLK)])

    return ek


_edge1 = _make_edge_kernel(W1PAD, H1)
_edge2 = _make_edge_kernel(W2PAD, NL, table_in_spmem=True)


def kernel(x, edge_index_all, W1, a1s, a1d, b1, g1, bt1, W2, a2s, a2d, b2, g2, bt2):
    src = jnp.pad(edge_index_all[0].reshape(NTILES, EPT),
                  ((0, 0), (0, EPT_PAD - EPT))).reshape(NTILES * NBLK, BLK)
    dst = jnp.pad(edge_index_all[1].reshape(NTILES, EPT),
                  ((0, 0), (0, EPT_PAD - EPT)),
                  constant_values=N).reshape(NTILES * NBLK, BLK)

    table1, s1, d1, m1 = _phase1(
        x, W1, a1s.reshape(H1, 1), a1d.reshape(H1, 1),
        g1.reshape(1, F_IN), bt1.reshape(1, F_IN))

    slab1 = _edge1(table1, s1.reshape(N), d1.reshape(N), m1.reshape(16), src, dst)

    table2, s2, d2, m2 = _phase2(
        slab1, b1.reshape(1, H1), g2.reshape(1, H1), bt2.reshape(1, H1),
        W2, a2s.reshape(NL, 1), a2d.reshape(NL, 1))

    slab2 = _edge2(table2, s2.reshape(N), d2.reshape(N), m2.reshape(16), src, dst)

    return _phase3(slab2, b2.reshape(1, NL))


# speedup vs baseline: 1.9176x; 1.8159x over previous
import functools

import jax
import jax.numpy as jnp
from jax import lax
from jax.experimental import pallas as pl
from jax.experimental.pallas import tpu as pltpu
from jax.experimental.pallas import tpu_sc as plsc

N = 10000
E = 320000
F_IN = 128
H1 = 64
NL = 16

NTILES = 32
EPT = E // NTILES
BLK = 128
NBLK = 80
EPT_PAD = NBLK * BLK
R = 10240
W1PAD = 72
W2PAD = 24


def _phase1_body(x_ref, w1_ref, a1s_ref, a1d_ref, g1_ref, bt1_ref,
                 table_ref, s_ref, d_ref, m_ref):
    xv = x_ref[...]
    mu = jnp.mean(xv, axis=0, keepdims=True)
    var = jnp.mean((xv - mu) ** 2, axis=0, keepdims=True)
    xn = (xv - mu) * lax.rsqrt(var + 1e-5) * g1_ref[...] + bt1_ref[...]
    hw = lax.dot_general(xn, w1_ref[...], (((1,), (0,)), ((), ())),
                         preferred_element_type=jnp.float32)
    s = lax.dot_general(hw, a1s_ref[...], (((1,), (0,)), ((), ())),
                        preferred_element_type=jnp.float32)
    d = lax.dot_general(hw, a1d_ref[...], (((1,), (0,)), ((), ())),
                        preferred_element_type=jnp.float32)
    table_ref[...] = jnp.concatenate(
        [hw, s, jnp.zeros((N, W1PAD - H1 - 1), jnp.float32)], axis=1)
    s_ref[...] = s
    d_ref[...] = d
    m = jnp.max(s) + jnp.max(d)
    m = jnp.maximum(m, 0.2 * m)
    m_ref[...] = jnp.full((1, 16), m, jnp.float32)


def _phase2_body(slab_ref, b1_ref, g2_ref, bt2_ref, w2_ref, a2s_ref, a2d_ref,
                 table_ref, s_ref, d_ref, m_ref):
    acc = slab_ref[0, :N, :] + slab_ref[1, :N, :]
    num = acc[:, :H1]
    den = acc[:, H1:H1 + 1]
    h = jnp.tanh(num / (den + 1e-16) + b1_ref[...])
    mu = jnp.mean(h, axis=0, keepdims=True)
    var = jnp.mean((h - mu) ** 2, axis=0, keepdims=True)
    hn = (h - mu) * lax.rsqrt(var + 1e-5) * g2_ref[...] + bt2_ref[...]
    hw = lax.dot_general(hn, w2_ref[...], (((1,), (0,)), ((), ())),
                         preferred_element_type=jnp.float32)
    s = lax.dot_general(hw, a2s_ref[...], (((1,), (0,)), ((), ())),
                        preferred_element_type=jnp.float32)
    d = lax.dot_general(hw, a2d_ref[...], (((1,), (0,)), ((), ())),
                        preferred_element_type=jnp.float32)
    table_ref[...] = hw
    s_ref[...] = s
    d_ref[...] = d
    m = jnp.max(s) + jnp.max(d)
    m = jnp.maximum(m, 0.2 * m)
    m_ref[...] = jnp.full((1, 16), m, jnp.float32)


def _phase3_body(slab_ref, b2_ref, out_ref):
    acc = slab_ref[0, :N, :] + slab_ref[1, :N, :]
    num = acc[:, :NL]
    den = acc[:, NL:NL + 1]
    h = jnp.tanh(num / (den + 1e-16) + b2_ref[...])
    mx = jnp.max(h, axis=1, keepdims=True)
    lse = jnp.log(jnp.sum(jnp.exp(h - mx), axis=1, keepdims=True))
    out_ref[...] = h - mx - lse


_phase1 = pl.pallas_call(
    _phase1_body,
    out_shape=[
        jax.ShapeDtypeStruct((N, W1PAD), jnp.float32),
        jax.ShapeDtypeStruct((N, 1), jnp.float32),
        jax.ShapeDtypeStruct((N, 1), jnp.float32),
        jax.ShapeDtypeStruct((1, 16), jnp.float32),
    ],
)

_phase2 = pl.pallas_call(
    _phase2_body,
    out_shape=[
        jax.ShapeDtypeStruct((N, NL), jnp.float32),
        jax.ShapeDtypeStruct((N, 1), jnp.float32),
        jax.ShapeDtypeStruct((N, 1), jnp.float32),
        jax.ShapeDtypeStruct((1, 16), jnp.float32),
    ],
)

_phase3 = pl.pallas_call(
    _phase3_body,
    out_shape=jax.ShapeDtypeStruct((N, NL), jnp.float32),
)


BLK1 = 80
NBLK1 = 126
EPT_PAD1 = NBLK1 * BLK1


def _make_edge_kernel_spmem(width: int, nfeat: int):
    rows_per_tile = R // 16
    mesh = plsc.VectorSubcoreMesh(core_axis_name="c", subcore_axis_name="s")

    @functools.partial(
        pl.kernel,
        out_type=jax.ShapeDtypeStruct((2, R, width), jnp.float32),
        mesh=mesh,
        compiler_params=pltpu.CompilerParams(needs_layout_passes=False,
                                             use_tc_tiling_on_sc=False),
        scratch_types=[
            pltpu.VMEM((N,), jnp.float32),
            pltpu.VMEM((16,), jnp.float32),
            pltpu.VMEM((BLK1,), jnp.float32),
            pltpu.VMEM((NBLK1, BLK1), jnp.int32),
            [pltpu.VMEM((BLK1,), jnp.int32) for _ in range(3)],
            [pltpu.VMEM((BLK1,), jnp.int32) for _ in range(3)],
            [pltpu.VMEM((BLK1, width), jnp.float32) for _ in range(3)],
            pltpu.VMEM_SHARED((R, width), jnp.float32),
            pltpu.VMEM_SHARED((N, width), jnp.float32),
            [pltpu.SemaphoreType.DMA for _ in range(3)],
            [pltpu.SemaphoreType.DMA for _ in range(3)],
        ],
    )
    def ek(table_hbm, d_hbm, m_hbm, pk_hbm, out_hbm,
           d_v, m_v, wbuf, pkb, src_unp, dst_unp, rows, accum, table_s,
           gsem, ssem):
        c = lax.axis_index("c")
        sub = lax.axis_index("s")
        wid = sub * 2 + c

        pltpu.sync_copy(d_hbm, d_v)
        pltpu.sync_copy(m_hbm, m_v)
        pltpu.sync_copy(pk_hbm.at[pl.ds(wid * NBLK1, NBLK1)], pkb)

        zeros16 = jnp.zeros((16,), jnp.float32)
        zoffs = [j * 16 for j in range(width // 16)]
        if width % 16:
            zoffs.append(width - 16)

        def zero_body(i, _):
            for z in zoffs:
                rows[0][i, pl.ds(z, 16)] = zeros16
            return _

        lax.fori_loop(0, BLK1, zero_body, None)

        def zcopy_body(k, _):
            pltpu.sync_copy(rows[0],
                            accum.at[pl.ds(sub * rows_per_tile + k * BLK1, BLK1)])
            return _

        lax.fori_loop(0, rows_per_tile // BLK1, zcopy_body, None)
        npt = N // 16
        pltpu.sync_copy(table_hbm.at[pl.ds(sub * npt, npt)],
                        table_s.at[pl.ds(sub * npt, npt)])
        plsc.subcore_barrier()

        mv = m_v[...]
        lane = lax.iota(jnp.int32, 16)
        c_s = jnp.zeros((16,), jnp.int32) + nfeat

        def gather_of(b, u):
            return pltpu.make_async_copy(table_s.at[src_unp[u]], rows[u], gsem[u])

        def scatter_of(b, u):
            return pltpu.make_async_copy(rows[u], accum.at[dst_unp[u]], ssem[u])

        def unpack(b, u):
            def up_body(g, _):
                pk = pkb[b, pl.ds(g * 16, 16)]
                src_unp[u][pl.ds(g * 16, 16)] = lax.shift_right_logical(pk, 16)
                dst_unp[u][pl.ds(g * 16, 16)] = jnp.bitwise_and(pk, 0xFFFF)
                return _

            lax.fori_loop(0, BLK1 // 16, up_body, None)

        unpack(0, 0)
        unpack(1, 1)
        gather_of(0, 0).start()
        gather_of(1, 1).start()

        def half(b, u):
            gather_of(b, u).wait()

            def w_body(g, _):
                e16 = g * 16 + lane
                sv = plsc.load_gather(rows[u], [e16, c_s])
                dv = plsc.load_gather(d_v, [dst_unp[u][pl.ds(g * 16, 16)]])
                t = sv + dv
                t = jnp.maximum(t, 0.2 * t)
                w16 = jnp.exp(t - mv)
                wbuf[pl.ds(g * 16, 16)] = w16
                plsc.store_scatter(rows[u], [e16, c_s], w16)
                return _

            lax.fori_loop(0, BLK1 // 16, w_body, None)

            def mul_body(i, _):
                for uu in range(4):
                    e = i * 4 + uu
                    wv = plsc.load_gather(wbuf, [jnp.zeros((16,), jnp.int32) + e])
                    for j in range(nfeat // 16):
                        rows[u][e, pl.ds(j * 16, 16)] = rows[u][e, pl.ds(j * 16, 16)] * wv
                return _

            lax.fori_loop(0, BLK1 // 4, mul_body, None)
            pltpu.async_copy(rows[u], accum.at[dst_unp[u]], ssem[u], add=True)

            @pl.when(b >= 1)
            def _drain():
                scatter_of(b - 1, (u + 2) % 3).wait()

            @pl.when(b + 2 < NBLK1)
            def _prefetch():
                unpack(b + 2, (u + 2) % 3)
                gather_of(b + 2, (u + 2) % 3).start()

        def loop_body(i, _):
            for u in range(3):
                half(3 * i + u, u)
            return _

        lax.fori_loop(0, NBLK1 // 3, loop_body, None)
        scatter_of(NBLK1 - 1, (NBLK1 - 1) % 3).wait()
        plsc.subcore_barrier()

        for k in range(rows_per_tile // 128):
            r0 = sub * rows_per_tile + k * 128
            pltpu.sync_copy(accum.at[pl.ds(r0, 128)], out_hbm.at[c, pl.ds(r0, 128)])

    return ek


def _make_edge_kernel(width: int, gwidth: int, table_in_spmem: bool = False):
    ngv = gwidth // 16
    nvec = width // 16
    rows_per_tile = R // 16
    mesh = plsc.VectorSubcoreMesh(core_axis_name="c", subcore_axis_name="s")

    @functools.partial(
        pl.kernel,
        out_type=jax.ShapeDtypeStruct((2, R, width), jnp.float32),
        mesh=mesh,
        compiler_params=pltpu.CompilerParams(needs_layout_passes=False,
                                             use_tc_tiling_on_sc=False),
        scratch_types=[
            pltpu.VMEM((N,), jnp.float32),
            pltpu.VMEM((N,), jnp.float32),
            pltpu.VMEM((16,), jnp.float32),
            pltpu.VMEM((NBLK, BLK), jnp.int32),
            pltpu.VMEM((NBLK, BLK), jnp.int32),
            pltpu.VMEM((BLK,), jnp.float32),
            [pltpu.VMEM((BLK, gwidth), jnp.float32) for _ in range(2)],
            [pltpu.VMEM((BLK, width), jnp.float32) for _ in range(2)],
            pltpu.VMEM_SHARED((R, width), jnp.float32),
            (pltpu.VMEM_SHARED((N, gwidth), jnp.float32)
             if table_in_spmem else pltpu.VMEM((16,), jnp.float32)),
            [pltpu.SemaphoreType.DMA for _ in range(2)],
            [pltpu.SemaphoreType.DMA for _ in range(2)],
        ],
    )
    def ek(table_hbm, s_hbm, d_hbm, m_hbm, src_hbm, dst_hbm, out_hbm,
           s_v, d_v, m_v, srcb, dstb, wbuf, rows, obuf, accum, table_s, gsem, ssem):
        c = lax.axis_index("c")
        sub = lax.axis_index("s")
        wid = sub * 2 + c

        pltpu.sync_copy(s_hbm, s_v)
        pltpu.sync_copy(d_hbm, d_v)
        pltpu.sync_copy(m_hbm, m_v)
        pltpu.sync_copy(src_hbm.at[pl.ds(wid * NBLK, NBLK)], srcb)
        pltpu.sync_copy(dst_hbm.at[pl.ds(wid * NBLK, NBLK)], dstb)

        zeros16 = jnp.zeros((16,), jnp.float32)

        zoffs = [j * 16 for j in range(width // 16)]
        if width % 16:
            zoffs.append(width - 16)

        def zero_body(i, _):
            for z in zoffs:
                obuf[0][i, pl.ds(z, 16)] = zeros16
                obuf[1][i, pl.ds(z, 16)] = zeros16
            return _

        lax.fori_loop(0, BLK, zero_body, None)

        def zcopy_body(k, _):
            pltpu.sync_copy(obuf[0], accum.at[pl.ds(sub * rows_per_tile + k * BLK, BLK)])
            return _

        lax.fori_loop(0, rows_per_tile // BLK, zcopy_body, None)
        if table_in_spmem:
            npt = N // 16
            pltpu.sync_copy(table_hbm.at[pl.ds(sub * npt, npt)],
                            table_s.at[pl.ds(sub * npt, npt)])
        plsc.subcore_barrier()

        mv = m_v[...]
        lane = lax.iota(jnp.int32, 16)

        table_ref = table_s if table_in_spmem else table_hbm

        def gather_of(b, u):
            return (
                pltpu.make_async_copy(table_ref.at[srcb.at[b, pl.ds(0, 64)]],
                                      rows[u].at[pl.ds(0, 64)], gsem[u]),
                pltpu.make_async_copy(table_ref.at[srcb.at[b, pl.ds(64, 64)]],
                                      rows[u].at[pl.ds(64, 64)], gsem[u]),
            )

        def scatter_of(b, v):
            return pltpu.make_async_copy(obuf[v], accum.at[dstb.at[b]], ssem[v])

        for g in gather_of(0, 0) + gather_of(1, 1):
            g.start()

        def half(b, u):
            def w_body(g, _):
                si = srcb[b, pl.ds(g * 16, 16)]
                di = dstb[b, pl.ds(g * 16, 16)]
                t = plsc.load_gather(s_v, [si]) + plsc.load_gather(d_v, [di])
                t = jnp.maximum(t, 0.2 * t)
                wbuf[pl.ds(g * 16, 16)] = jnp.exp(t - mv)
                return _

            lax.fori_loop(0, BLK // 16, w_body, None)
            for g in gather_of(b, u):
                g.wait()

            @pl.when(b >= 2)
            def _drain():
                scatter_of(b - 2, u).wait()

            def wcol_body(g, _):
                w16 = wbuf[pl.ds(g * 16, 16)]
                plsc.store_scatter(obuf[u], [g * 16 + lane,
                                             jnp.zeros((16,), jnp.int32) + gwidth], w16)
                return _

            lax.fori_loop(0, BLK // 16, wcol_body, None)

            def mul_body(i, _):
                for uu in range(4):
                    e = i * 4 + uu
                    wv = plsc.load_gather(wbuf, [jnp.zeros((16,), jnp.int32) + e])
                    for j in range(ngv):
                        obuf[u][e, pl.ds(j * 16, 16)] = rows[u][e, pl.ds(j * 16, 16)] * wv
                return _

            lax.fori_loop(0, BLK // 4, mul_body, None)

            @pl.when(b + 2 < NBLK)
            def _prefetch():
                for g in gather_of(b + 2, u):
                    g.start()

            pltpu.async_copy(obuf[u], accum.at[dstb.at[b]], ssem[u], add=True)

        def loop_body(i, _):
            half(2 * i, 0)
            half(2 * i + 1, 1)
            return _

        lax.fori_loop(0, NBLK // 2, loop_body, None)
        scatter_of(NBLK - 2, 0).wait()
        scatter_of(NBLK - 1, 1).wait()
        plsc.subcore_barrier()

        for k in range(rows_per_tile // BLK):
            r0 = sub * rows_per_tile + k * BLK
            pltpu.sync_copy(accum.at[pl.ds(r0, BLK)], out_hbm.at[c, pl.ds(r0, BLK)])

    return ek


_edge1 = _make_edge_kernel_spmem(W1PAD, H1)
_edge2 = _make_edge_kernel(W2PAD, NL, table_in_spmem=True)


def kernel(x, edge_index_all, W1, a1s, a1d, b1, g1, bt1, W2, a2s, a2d, b2, g2, bt2):
    src = jnp.pad(edge_index_all[0].reshape(NTILES, EPT),
                  ((0, 0), (0, EPT_PAD - EPT))).reshape(NTILES * NBLK, BLK)
    dst = jnp.pad(edge_index_all[1].reshape(NTILES, EPT),
                  ((0, 0), (0, EPT_PAD - EPT)),
                  constant_values=N).reshape(NTILES * NBLK, BLK)
    packed = jnp.pad(
        (edge_index_all[0] << 16 | edge_index_all[1]).reshape(NTILES, EPT),
        ((0, 0), (0, EPT_PAD1 - EPT)),
        constant_values=N).reshape(NTILES * NBLK1, BLK1)

    table1, s1, d1, m1 = _phase1(
        x, W1, a1s.reshape(H1, 1), a1d.reshape(H1, 1),
        g1.reshape(1, F_IN), bt1.reshape(1, F_IN))

    slab1 = _edge1(table1, d1.reshape(N), m1.reshape(16), packed)

    table2, s2, d2, m2 = _phase2(
        slab1, b1.reshape(1, H1), g2.reshape(1, H1), bt2.reshape(1, H1),
        W2, a2s.reshape(NL, 1), a2d.reshape(NL, 1))

    slab2 = _edge2(table2, s2.reshape(N), d2.reshape(N), m2.reshape(16), src, dst)

    return _phase3(slab2, b2.reshape(1, NL))


# speedup vs baseline: 1.9728x; 1.0288x over previous
import functools

import jax
import jax.numpy as jnp
from jax import lax
from jax.experimental import pallas as pl
from jax.experimental.pallas import tpu as pltpu
from jax.experimental.pallas import tpu_sc as plsc

N = 10000
E = 320000
F_IN = 128
H1 = 64
NL = 16

NTILES = 32
EPT = E // NTILES
BLK = 128
NBLK = 80
EPT_PAD = NBLK * BLK
R = 10240
W1PAD = 72
W2PAD = 24


def _phase1_body(x_ref, w1_ref, a1s_ref, a1d_ref, g1_ref, bt1_ref,
                 table_ref, s_ref, d_ref, m_ref):
    xv = x_ref[...]
    mu = jnp.mean(xv, axis=0, keepdims=True)
    var = jnp.mean((xv - mu) ** 2, axis=0, keepdims=True)
    xn = (xv - mu) * lax.rsqrt(var + 1e-5) * g1_ref[...] + bt1_ref[...]
    hw = lax.dot_general(xn, w1_ref[...], (((1,), (0,)), ((), ())),
                         preferred_element_type=jnp.float32)
    s = lax.dot_general(hw, a1s_ref[...], (((1,), (0,)), ((), ())),
                        preferred_element_type=jnp.float32)
    d = lax.dot_general(hw, a1d_ref[...], (((1,), (0,)), ((), ())),
                        preferred_element_type=jnp.float32)
    table_ref[...] = jnp.concatenate(
        [hw, s, jnp.zeros((N, W1PAD - H1 - 1), jnp.float32)], axis=1)
    s_ref[...] = s
    d_ref[...] = d
    m = jnp.max(s) + jnp.max(d)
    m = jnp.maximum(m, 0.2 * m)
    m_ref[...] = jnp.full((1, 16), m, jnp.float32)


def _phase2_body(slab_ref, b1_ref, g2_ref, bt2_ref, w2_ref, a2s_ref, a2d_ref,
                 table_ref, s_ref, d_ref, m_ref):
    acc = slab_ref[0, :N, :] + slab_ref[1, :N, :]
    num = acc[:, :H1]
    den = acc[:, H1:H1 + 1]
    h = jnp.tanh(num / (den + 1e-16) + b1_ref[...])
    mu = jnp.mean(h, axis=0, keepdims=True)
    var = jnp.mean((h - mu) ** 2, axis=0, keepdims=True)
    hn = (h - mu) * lax.rsqrt(var + 1e-5) * g2_ref[...] + bt2_ref[...]
    hw = lax.dot_general(hn, w2_ref[...], (((1,), (0,)), ((), ())),
                         preferred_element_type=jnp.float32)
    s = lax.dot_general(hw, a2s_ref[...], (((1,), (0,)), ((), ())),
                        preferred_element_type=jnp.float32)
    d = lax.dot_general(hw, a2d_ref[...], (((1,), (0,)), ((), ())),
                        preferred_element_type=jnp.float32)
    table_ref[...] = jnp.concatenate(
        [hw, s, jnp.zeros((N, W2PAD - NL - 1), jnp.float32)], axis=1)
    s_ref[...] = s
    d_ref[...] = d
    m = jnp.max(s) + jnp.max(d)
    m = jnp.maximum(m, 0.2 * m)
    m_ref[...] = jnp.full((1, 16), m, jnp.float32)


def _phase3_body(slab_ref, b2_ref, out_ref):
    acc = slab_ref[0, :N, :] + slab_ref[1, :N, :]
    num = acc[:, :NL]
    den = acc[:, NL:NL + 1]
    h = jnp.tanh(num / (den + 1e-16) + b2_ref[...])
    mx = jnp.max(h, axis=1, keepdims=True)
    lse = jnp.log(jnp.sum(jnp.exp(h - mx), axis=1, keepdims=True))
    out_ref[...] = h - mx - lse


_phase1 = pl.pallas_call(
    _phase1_body,
    out_shape=[
        jax.ShapeDtypeStruct((N, W1PAD), jnp.float32),
        jax.ShapeDtypeStruct((N, 1), jnp.float32),
        jax.ShapeDtypeStruct((N, 1), jnp.float32),
        jax.ShapeDtypeStruct((1, 16), jnp.float32),
    ],
)

_phase2 = pl.pallas_call(
    _phase2_body,
    out_shape=[
        jax.ShapeDtypeStruct((N, W2PAD), jnp.float32),
        jax.ShapeDtypeStruct((N, 1), jnp.float32),
        jax.ShapeDtypeStruct((N, 1), jnp.float32),
        jax.ShapeDtypeStruct((1, 16), jnp.float32),
    ],
)

_phase3 = pl.pallas_call(
    _phase3_body,
    out_shape=jax.ShapeDtypeStruct((N, NL), jnp.float32),
)


BLK1 = 80
NBLK1 = 126
EPT_PAD1 = NBLK1 * BLK1


def _make_edge_kernel_spmem(width: int, nfeat: int):
    rows_per_tile = R // 16
    mesh = plsc.VectorSubcoreMesh(core_axis_name="c", subcore_axis_name="s")

    @functools.partial(
        pl.kernel,
        out_type=jax.ShapeDtypeStruct((2, R, width), jnp.float32),
        mesh=mesh,
        compiler_params=pltpu.CompilerParams(needs_layout_passes=False,
                                             use_tc_tiling_on_sc=False),
        scratch_types=[
            pltpu.VMEM((N,), jnp.float32),
            pltpu.VMEM((16,), jnp.float32),
            pltpu.VMEM((BLK1,), jnp.float32),
            pltpu.VMEM((NBLK1, BLK1), jnp.int32),
            [pltpu.VMEM((BLK1,), jnp.int32) for _ in range(3)],
            [pltpu.VMEM((BLK1,), jnp.int32) for _ in range(3)],
            [pltpu.VMEM((BLK1, width), jnp.float32) for _ in range(3)],
            pltpu.VMEM_SHARED((R, width), jnp.float32),
            pltpu.VMEM_SHARED((N, width), jnp.float32),
            [pltpu.SemaphoreType.DMA for _ in range(3)],
            [pltpu.SemaphoreType.DMA for _ in range(3)],
        ],
    )
    def ek(table_hbm, d_hbm, m_hbm, pk_hbm, out_hbm,
           d_v, m_v, wbuf, pkb, src_unp, dst_unp, rows, accum, table_s,
           gsem, ssem):
        c = lax.axis_index("c")
        sub = lax.axis_index("s")
        wid = sub * 2 + c

        pltpu.sync_copy(d_hbm, d_v)
        pltpu.sync_copy(m_hbm, m_v)
        pltpu.sync_copy(pk_hbm.at[pl.ds(wid * NBLK1, NBLK1)], pkb)

        zeros16 = jnp.zeros((16,), jnp.float32)
        zoffs = [j * 16 for j in range(width // 16)]
        if width % 16:
            zoffs.append(width - 16)

        def zero_body(i, _):
            for z in zoffs:
                rows[0][i, pl.ds(z, 16)] = zeros16
            return _

        lax.fori_loop(0, BLK1, zero_body, None)

        def zcopy_body(k, _):
            pltpu.sync_copy(rows[0],
                            accum.at[pl.ds(sub * rows_per_tile + k * BLK1, BLK1)])
            return _

        lax.fori_loop(0, rows_per_tile // BLK1, zcopy_body, None)
        npt = N // 16
        pltpu.sync_copy(table_hbm.at[pl.ds(sub * npt, npt)],
                        table_s.at[pl.ds(sub * npt, npt)])
        plsc.subcore_barrier()

        mv = m_v[...]
        lane = lax.iota(jnp.int32, 16)
        c_s = jnp.zeros((16,), jnp.int32) + nfeat

        def gather_of(b, u):
            return pltpu.make_async_copy(table_s.at[src_unp[u]], rows[u], gsem[u])

        def scatter_of(b, u):
            return pltpu.make_async_copy(rows[u], accum.at[dst_unp[u]], ssem[u])

        def unpack(b, u):
            def up_body(g, _):
                pk = pkb[b, pl.ds(g * 16, 16)]
                src_unp[u][pl.ds(g * 16, 16)] = lax.shift_right_logical(pk, 16)
                dst_unp[u][pl.ds(g * 16, 16)] = jnp.bitwise_and(pk, 0xFFFF)
                return _

            lax.fori_loop(0, BLK1 // 16, up_body, None)

        unpack(0, 0)
        unpack(1, 1)
        gather_of(0, 0).start()
        gather_of(1, 1).start()

        def half(b, u):
            gather_of(b, u).wait()

            def w_body(g, _):
                e16 = g * 16 + lane
                sv = plsc.load_gather(rows[u], [e16, c_s])
                dv = plsc.load_gather(d_v, [dst_unp[u][pl.ds(g * 16, 16)]])
                t = sv + dv
                t = jnp.maximum(t, 0.2 * t)
                w16 = jnp.exp(t - mv)
                wbuf[pl.ds(g * 16, 16)] = w16
                plsc.store_scatter(rows[u], [e16, c_s], w16)
                return _

            lax.fori_loop(0, BLK1 // 16, w_body, None)

            def mul_body(i, _):
                for uu in range(4):
                    e = i * 4 + uu
                    wv = plsc.load_gather(wbuf, [jnp.zeros((16,), jnp.int32) + e])
                    for j in range(nfeat // 16):
                        rows[u][e, pl.ds(j * 16, 16)] = rows[u][e, pl.ds(j * 16, 16)] * wv
                return _

            lax.fori_loop(0, BLK1 // 4, mul_body, None)
            pltpu.async_copy(rows[u], accum.at[dst_unp[u]], ssem[u], add=True)

            @pl.when(b >= 1)
            def _drain():
                scatter_of(b - 1, (u + 2) % 3).wait()

            @pl.when(b + 2 < NBLK1)
            def _prefetch():
                unpack(b + 2, (u + 2) % 3)
                gather_of(b + 2, (u + 2) % 3).start()

        def loop_body(i, _):
            for u in range(3):
                half(3 * i + u, u)
            return _

        lax.fori_loop(0, NBLK1 // 3, loop_body, None)
        scatter_of(NBLK1 - 1, (NBLK1 - 1) % 3).wait()
        plsc.subcore_barrier()

        for k in range(rows_per_tile // 128):
            r0 = sub * rows_per_tile + k * 128
            pltpu.sync_copy(accum.at[pl.ds(r0, 128)], out_hbm.at[c, pl.ds(r0, 128)])

    return ek


_edge1 = _make_edge_kernel_spmem(W1PAD, H1)
_edge2 = _make_edge_kernel_spmem(W2PAD, NL)


def kernel(x, edge_index_all, W1, a1s, a1d, b1, g1, bt1, W2, a2s, a2d, b2, g2, bt2):
    packed = jnp.pad(
        (edge_index_all[0] << 16 | edge_index_all[1]).reshape(NTILES, EPT),
        ((0, 0), (0, EPT_PAD1 - EPT)),
        constant_values=N).reshape(NTILES * NBLK1, BLK1)

    table1, s1, d1, m1 = _phase1(
        x, W1, a1s.reshape(H1, 1), a1d.reshape(H1, 1),
        g1.reshape(1, F_IN), bt1.reshape(1, F_IN))

    slab1 = _edge1(table1, d1.reshape(N), m1.reshape(16), packed)

    table2, s2, d2, m2 = _phase2(
        slab1, b1.reshape(1, H1), g2.reshape(1, H1), bt2.reshape(1, H1),
        W2, a2s.reshape(NL, 1), a2d.reshape(NL, 1))

    slab2 = _edge2(table2, d2.reshape(N), m2.reshape(16), packed)

    return _phase3(slab2, b2.reshape(1, NL))
